# baseline (speedup 1.0000x reference)
import functools

import jax
import jax.numpy as jnp
from jax import lax
from jax.experimental import pallas as pl
from jax.experimental.pallas import tpu as pltpu
from jax.experimental.pallas import tpu_sc as plsc

NC = 2
NS = 16
L = 16

N1 = 12000
N2 = 10000
R = 8
NB = 4
D_IN = 200
D_H = 128
D_F = 200
B = 512
S = 64

_C_EPT = 12288
_C_E1P = _C_EPT * NS
_C_NSB = _C_EPT // 256
_C_ROWS = 12160
_C_RPT = _C_ROWS // NS
_C_W = 64
_C_NP = 4

_G_EPT = 20480
_G_E2P = _G_EPT * NS
_G_NSB = _G_EPT // 256
_G_EPW = _G_E2P // (NC * NS)
_G_NSBW = _G_EPW // 256
_G_ROWS = 80128
_G_RPT = _G_ROWS // NS
_G_W = 16


def _mesh():
    return plsc.VectorSubcoreMesh(core_axis_name="c", subcore_axis_name="s",
                                  num_cores=NC, num_subcores=NS)


def _concept_sc(xt, srcp, dstp, z2d, z16, ones16):
    ppc = _C_NP // NC
    nst = _C_NSB
    half = nst // 2

    @functools.partial(
        pl.kernel,
        out_type=(jax.ShapeDtypeStruct((_C_NP, _C_ROWS, _C_W), jnp.float32),
                  jax.ShapeDtypeStruct((_C_ROWS, 16), jnp.float32)),
        mesh=_mesh(),
        compiler_params=pltpu.CompilerParams(use_tc_tiling_on_sc=False),
        scratch_types=[
            pltpu.VMEM((_C_NSB, 256), jnp.int32),
            pltpu.VMEM((_C_NSB, 256), jnp.int32),
            pltpu.VMEM((256, _C_W), jnp.float32),
            pltpu.VMEM((256, _C_W), jnp.float32),
            pltpu.VMEM((256, 16), jnp.float32),
            pltpu.VMEM_SHARED((_C_ROWS, _C_W), jnp.float32),
            pltpu.VMEM_SHARED((_C_ROWS, 16), jnp.float32),
            pltpu.SemaphoreType.DMA,
            pltpu.SemaphoreType.DMA,
        ],
    )
    def k(xt_h, srcp_h, dstp_h, z2d_h, z16_h, ones_h, out_agg, out_deg,
          segb, gidx, rowsa, rowsb, ones_v, acc, deg, sema, semb):
        c = lax.axis_index("c")
        s = lax.axis_index("s")
        pltpu.sync_copy(ones_h, ones_v)
        r0 = s * _C_RPT
        pltpu.sync_copy(dstp_h.at[pl.ds(s * nst, nst)], segb)

        @pl.when(c == 0)
        def _():
            pltpu.sync_copy(z16_h.at[pl.ds(r0, _C_RPT)], deg.at[pl.ds(r0, _C_RPT)])

        for j in range(ppc):
            p = c * ppc + j
            pltpu.sync_copy(z2d_h.at[pl.ds(r0, _C_RPT)], acc.at[pl.ds(r0, _C_RPT)])
            pltpu.sync_copy(srcp_h.at[pl.ds(s * nst, nst)], gidx)

            def gstep(st, carry):
                for i in range(256 // L):
                    sl = pl.ds(i * L, L)
                    gidx[st, sl] = gidx[st, sl] * _C_NP + p
                return carry

            lax.fori_loop(0, nst, gstep, 0)
            plsc.subcore_barrier()
            pltpu.async_copy(xt_h.at[gidx.at[0]], rowsa, sema)

            def pair(t, carry):
                i0 = 2 * t
                pltpu.async_copy(xt_h.at[gidx.at[i0 + 1]], rowsb, semb)
                pltpu.make_async_copy(xt_h.at[gidx.at[i0]], rowsa, sema).wait()
                pltpu.sync_copy(rowsa, acc.at[segb.at[i0]], add=True)

                @pl.when(t < half - 1)
                def _():
                    pltpu.async_copy(xt_h.at[gidx.at[i0 + 2]], rowsa, sema)

                pltpu.make_async_copy(xt_h.at[gidx.at[i0 + 1]], rowsb, semb).wait()
                pltpu.sync_copy(rowsb, acc.at[segb.at[i0 + 1]], add=True)
                if j == 0:
                    @pl.when(c == 0)
                    def _():
                        pltpu.sync_copy(ones_v, deg.at[segb.at[i0]], add=True)
                        pltpu.sync_copy(ones_v, deg.at[segb.at[i0 + 1]], add=True)
                return carry

            lax.fori_loop(0, half, pair, 0)
            plsc.subcore_barrier()
            pltpu.sync_copy(acc.at[pl.ds(r0, _C_RPT)],
                            out_agg.at[p, pl.ds(r0, _C_RPT)])

        @pl.when(c == 0)
        def _():
            pltpu.sync_copy(deg.at[pl.ds(r0, _C_RPT)], out_deg.at[pl.ds(r0, _C_RPT)])

    return k(xt, srcp, dstp, z2d, z16, ones16)


def _cnt_sc(dstp, typep, z16, ones16):
    @functools.partial(
        pl.kernel,
        out_type=jax.ShapeDtypeStruct((NC, _G_ROWS, 16), jnp.float32),
        mesh=_mesh(),
        compiler_params=pltpu.CompilerParams(use_tc_tiling_on_sc=False),
        scratch_types=[
            pltpu.VMEM((_G_NSBW, 256), jnp.int32),
            pltpu.VMEM((_G_NSBW, 256), jnp.int32),
            pltpu.VMEM((256, 16), jnp.float32),
            pltpu.VMEM_SHARED((_G_ROWS, 16), jnp.float32),
        ],
    )
    def k(dstp_h, typep_h, z16_h, ones_h, out_cnt, segb, tyb, ones_v, cntb):
        c = lax.axis_index("c")
        s = lax.axis_index("s")
        pltpu.sync_copy(ones_h, ones_v)
        g0 = s * _G_RPT
        pltpu.sync_copy(z16_h.at[pl.ds(g0, _G_RPT)], cntb.at[pl.ds(g0, _G_RPT)])
        wrow = (s * NC + c) * _G_NSBW
        pltpu.sync_copy(dstp_h.at[pl.ds(wrow, _G_NSBW)], segb)
        pltpu.sync_copy(typep_h.at[pl.ds(wrow, _G_NSBW)], tyb)

        def prep(st, carry):
            for i in range(256 // L):
                sl = pl.ds(i * L, L)
                segb[st, sl] = tyb[st, sl] * N2 + segb[st, sl]
            return carry

        lax.fori_loop(0, _G_NSBW, prep, 0)
        plsc.subcore_barrier()

        def cstep(st, carry):
            pltpu.sync_copy(ones_v, cntb.at[segb.at[st]], add=True)
            return carry

        lax.fori_loop(0, _G_NSBW, cstep, 0)
        plsc.subcore_barrier()
        pltpu.sync_copy(cntb.at[pl.ds(g0, _G_RPT)], out_cnt.at[c, pl.ds(g0, _G_RPT)])

    return k(dstp, typep, z16, ones16)


def _rgcn_sc(xt, srcp, dstp, typep, z16, np16):
    ppc = np16 // NC
    nst = _G_NSB
    half = nst // 2

    @functools.partial(
        pl.kernel,
        out_type=jax.ShapeDtypeStruct((np16, _G_ROWS, _G_W), jnp.float32),
        mesh=_mesh(),
        compiler_params=pltpu.CompilerParams(use_tc_tiling_on_sc=False),
        scratch_types=[
            pltpu.VMEM((nst, 256), jnp.int32),
            pltpu.VMEM((nst, 256), jnp.int32),
            pltpu.VMEM((256, _G_W), jnp.float32),
            pltpu.VMEM((256, _G_W), jnp.float32),
            pltpu.VMEM_SHARED((_G_ROWS, _G_W), jnp.float32),
            pltpu.SemaphoreType.DMA,
            pltpu.SemaphoreType.DMA,
            pltpu.SemaphoreType.DMA,
            pltpu.SemaphoreType.DMA,
        ],
    )
    def k(xt_h, srcp_h, dstp_h, typep_h, z16_h,
          out_agg, segb, gidx, rowsa, rowsb, acc, sema, semb, semsa, semsb):
        c = lax.axis_index("c")
        s = lax.axis_index("s")
        r0 = s * _G_RPT
        pltpu.sync_copy(dstp_h.at[pl.ds(s * nst, nst)], segb)
        pltpu.sync_copy(typep_h.at[pl.ds(s * nst, nst)], gidx)

        def segstep(st, carry):
            for i in range(256 // L):
                sl = pl.ds(i * L, L)
                segb[st, sl] = gidx[st, sl] * N2 + segb[st, sl]
            return carry

        lax.fori_loop(0, nst, segstep, 0)

        for j in range(ppc):
            p = c * ppc + j
            pltpu.sync_copy(z16_h.at[pl.ds(r0, _G_RPT)], acc.at[pl.ds(r0, _G_RPT)])
            pltpu.sync_copy(srcp_h.at[pl.ds(s * nst, nst)], gidx)

            def gstep(st, carry):
                for i in range(256 // L):
                    sl = pl.ds(i * L, L)
                    gidx[st, sl] = gidx[st, sl] * np16 + p
                return carry

            lax.fori_loop(0, nst, gstep, 0)
            plsc.subcore_barrier()

            pltpu.async_copy(xt_h.at[gidx.at[0]], rowsa, sema)
            pltpu.async_copy(xt_h.at[gidx.at[1]], rowsb, semb)

            def pair(t, carry):
                i0 = 2 * t
                pltpu.make_async_copy(xt_h.at[gidx.at[i0]], rowsa, sema).wait()
                pltpu.async_copy(rowsa, acc.at[segb.at[i0]], semsa, add=True)
                pltpu.make_async_copy(xt_h.at[gidx.at[i0 + 1]], rowsb, semb).wait()
                pltpu.async_copy(rowsb, acc.at[segb.at[i0 + 1]], semsb, add=True)
                pltpu.make_async_copy(rowsa, acc.at[segb.at[i0]], semsa).wait()
                pltpu.make_async_copy(rowsb, acc.at[segb.at[i0 + 1]], semsb).wait()

                @pl.when(t < half - 1)
                def _():
                    pltpu.async_copy(xt_h.at[gidx.at[i0 + 2]], rowsa, sema)
                    pltpu.async_copy(xt_h.at[gidx.at[i0 + 3]], rowsb, semb)

                return carry

            lax.fori_loop(0, half, pair, 0)
            plsc.subcore_barrier()
            pltpu.sync_copy(acc.at[pl.ds(r0, _G_RPT)],
                            out_agg.at[p, pl.ds(r0, _G_RPT)])

    return k(xt, srcp, dstp, typep, z16)


def _pad_edges_g2(edge_index, edge_type):
    pad = _G_E2P - edge_index.shape[1]
    ar = jnp.arange(pad, dtype=jnp.int32)
    srcp = jnp.concatenate([edge_index[0], jnp.zeros((pad,), jnp.int32)])
    dstp = jnp.concatenate([edge_index[1], ar % 128])
    typep = jnp.concatenate([edge_type, jnp.full((pad,), R, jnp.int32)])
    return (srcp.reshape(-1, 256), dstp.reshape(-1, 256),
            typep.reshape(-1, 256))


def _combine_bases(comb, bases, kp):
    kk, mm = bases.shape[1], bases.shape[2]

    def body(comb_ref, bases_ref, out_ref):
        cb = comb_ref[...]
        bs = bases_ref[...].reshape(NB, kk * mm)
        wf = jnp.dot(cb, bs, preferred_element_type=jnp.float32).reshape(R, kk, mm)
        if kp > kk:
            wf = jnp.concatenate(
                [wf, jnp.zeros((R, kp - kk, mm), jnp.float32)], axis=1)
        out_ref[...] = wf

    return pl.pallas_call(
        body, out_shape=jax.ShapeDtypeStruct((R, kp, mm), jnp.float32),
    )(comb, bases)


def _rgcn_dense(agg, cnt, x, wk, root, bias, relu, np16, mp):
    kk = np16 * _G_W
    mm = wk.shape[2]
    bn = 400
    nblk = N2 // bn

    def body(agg_ref, cnt_ref, x_ref, wk_ref, root_ref, bias_ref, out_ref):
        rr = pl.program_id(1)

        @pl.when(rr < R)
        def _():
            ab = agg_ref[...]
            a = jnp.concatenate([ab[i] for i in range(np16)], axis=1)
            cv = cnt_ref[...]
            a = a / jnp.maximum(cv[0, :, 0:1] + cv[1, :, 0:1], 1.0)
            t = jnp.dot(a, wk_ref[...][0], preferred_element_type=jnp.float32)

            @pl.when(rr == 0)
            def _():
                if mp > mm:
                    out_ref[...] = jnp.concatenate(
                        [t, jnp.zeros((bn, mp - mm), jnp.float32)], axis=1)
                else:
                    out_ref[...] = t

            @pl.when(rr > 0)
            def _():
                out_ref[:, :mm] += t

        @pl.when(rr == R)
        def _():
            t = jnp.dot(x_ref[...], root_ref[...],
                        preferred_element_type=jnp.float32) + bias_ref[...]
            acc2 = out_ref[:, :mm] + t
            if relu:
                acc2 = jax.nn.relu(acc2)
            out_ref[:, :mm] = acc2

    return pl.pallas_call(
        body,
        grid=(nblk, R + 1),
        in_specs=[
            pl.BlockSpec((np16, bn, _G_W),
                         lambda nb, rr: (0, jnp.minimum(rr, R - 1) * (N2 // 400) + nb, 0)),
            pl.BlockSpec((NC, bn, 16),
                         lambda nb, rr: (0, jnp.minimum(rr, R - 1) * (N2 // 400) + nb, 0)),
            pl.BlockSpec((bn, x.shape[1]), lambda nb, rr: (nb, 0)),
            pl.BlockSpec((1, kk, mm), lambda nb, rr: (jnp.minimum(rr, R - 1), 0, 0)),
            pl.BlockSpec((x.shape[1], mm), lambda nb, rr: (0, 0)),
            pl.BlockSpec((1, mm), lambda nb, rr: (0, 0)),
        ],
        out_specs=pl.BlockSpec((bn, mp), lambda nb, rr: (nb, 0)),
        out_shape=jax.ShapeDtypeStruct((N2, mp), jnp.float32),
    )(agg, cnt, x, wk, root, bias.reshape(1, mm))


def _norm_concept(agg, deg):
    bn = 480
    nblk = N1 // bn

    def body(agg_ref, deg_ref, out1_ref, out2_ref):
        ab = agg_ref[...]
        d = jnp.maximum(deg_ref[...][:, 0:1], 1.0)
        xv = jnp.concatenate([ab[i] for i in range(_C_NP)], axis=1)[:, :D_IN]
        xv = jax.nn.relu(xv / d)
        out1_ref[...] = xv
        out2_ref[...] = jnp.concatenate(
            [jnp.clip(xv, 0.0, 1.0), jnp.zeros((bn, 8), jnp.float32)], axis=1)

    return pl.pallas_call(
        body,
        grid=(nblk,),
        in_specs=[
            pl.BlockSpec((_C_NP, bn, _C_W), lambda nb: (0, nb, 0)),
            pl.BlockSpec((bn, 16), lambda nb: (nb, 0)),
        ],
        out_specs=[
            pl.BlockSpec((bn, D_IN), lambda nb: (nb, 0)),
            pl.BlockSpec((bn, 208), lambda nb: (nb, 0)),
        ],
        out_shape=[
            jax.ShapeDtypeStruct((N1, D_IN), jnp.float32),
            jax.ShapeDtypeStruct((N1, 208), jnp.float32),
        ],
    )(agg, deg)


def _concept_layer(x, edge_index):
    src = edge_index[0]
    dst = edge_index[1]
    pad = _C_E1P - src.shape[0]
    srcp = jnp.concatenate([src, jnp.zeros((pad,), jnp.int32)]).reshape(-1, 256)
    dstp = jnp.concatenate(
        [dst, N1 + (jnp.arange(pad, dtype=jnp.int32) % 128)]).reshape(-1, 256)
    xt = jnp.pad(x, ((0, 0), (0, _C_NP * _C_W - D_IN))).reshape(N1 * _C_NP, _C_W)
    z2d = jnp.zeros((_C_ROWS, _C_W), jnp.float32)
    z16c = jnp.zeros((_C_ROWS, 16), jnp.float32)
    ones16 = jnp.ones((256, 16), jnp.float32)
    agg, deg = _concept_sc(xt, srcp, dstp, z2d, z16c, ones16)
    return _norm_concept(agg[:, :N1], deg[:N1])


def _gather_sc(xg2p, xclip, ilist, sflat):
    spt = (B * S) // (NC * NS)
    nstr = spt // 128
    ept = B // (NC * NS)

    @functools.partial(
        pl.kernel,
        out_type=(jax.ShapeDtypeStruct((B, 208), jnp.float32),
                  jax.ShapeDtypeStruct((B * S, 208), jnp.float32)),
        mesh=_mesh(),
        compiler_params=pltpu.CompilerParams(use_tc_tiling_on_sc=False),
        scratch_types=[
            pltpu.VMEM((1, 128), jnp.int32),
            pltpu.VMEM((1, ept), jnp.int32),
            pltpu.VMEM((128, 208), jnp.float32),
            pltpu.VMEM((ept, 208), jnp.float32),
            pltpu.SemaphoreType.DMA,
        ],
    )
    def k(xg2_h, xclip_h, ilist_h, sflat_h, out_en, out_sam,
          sidx, eidx, rows, rows_e, sem):
        c = lax.axis_index("c")
        s = lax.axis_index("s")
        wid = s * NC + c
        pltpu.sync_copy(ilist_h.at[pl.ds(wid * ept, ept)], eidx.at[0])
        pltpu.async_copy(xg2_h.at[eidx.at[0]], rows_e, sem).wait()
        pltpu.sync_copy(rows_e, out_en.at[pl.ds(wid * ept, ept)])

        def step(st, carry):
            off = wid * spt + st * 128
            pltpu.sync_copy(sflat_h.at[pl.ds(off, 128)], sidx.at[0])
            pltpu.async_copy(xclip_h.at[sidx.at[0]], rows, sem).wait()
            pltpu.sync_copy(rows, out_sam.at[pl.ds(off, 128)])
            return carry

        lax.fori_loop(0, nstr, step, 0)

    return k(xg2p, xclip, ilist, sflat)


def _predict_body(en_ref, sam_ref, w_ref, out_ref):
    en = en_ref[...]
    sam = sam_ref[...]
    w = jnp.clip(w_ref[...], 0.0, 1.0)
    v = en * en * w
    tmp = lax.dot_general(sam, v, (((2,), (1,)), ((0,), (0,))),
                          preferred_element_type=jnp.float32)
    m = jnp.max(tmp, axis=1, keepdims=True)
    e = jnp.exp(tmp - m)
    out_ref[...] = e / jnp.sum(e, axis=1, keepdims=True)


def _predict(en, sam, wp):
    bb = 64
    return pl.pallas_call(
        _predict_body,
        grid=(B // bb,),
        in_specs=[
            pl.BlockSpec((bb, 208), lambda i: (i, 0)),
            pl.BlockSpec((bb, S, 208), lambda i: (i, 0, 0)),
            pl.BlockSpec((1, 208), lambda i: (0, 0)),
        ],
        out_specs=pl.BlockSpec((bb, S), lambda i: (i, 0)),
        out_shape=jax.ShapeDtypeStruct((B, S), jnp.float32),
    )(en, sam, wp)


def kernel(all_node_embedding, edge_index_g2, edge_type_g2, edge_index_g1, index_list, sample_index,
           bases1, comb1, root1, bias1, bases2, comb2, root2, bias2, weights):
    x_g1, xclip = _concept_layer(all_node_embedding, edge_index_g1)

    srcp2, dstp2, typep2 = _pad_edges_g2(edge_index_g2, edge_type_g2)
    z16 = jnp.zeros((_G_ROWS, 16), jnp.float32)
    ones16 = jnp.ones((256, 16), jnp.float32)
    cnt = _cnt_sc(dstp2, typep2, z16, ones16)

    x2 = x_g1[:N2]
    xt1 = jnp.pad(x2, ((0, 0), (0, 24))).reshape(N2 * 14, _G_W)
    agg1 = _rgcn_sc(xt1, srcp2, dstp2, typep2, z16, 14)
    wk1 = _combine_bases(comb1, bases1, 224)
    h1 = _rgcn_dense(agg1, cnt, x2, wk1, root1, bias1, True, 14, D_H)

    xt2 = h1.reshape(N2 * 8, _G_W)
    agg2 = _rgcn_sc(xt2, srcp2, dstp2, typep2, z16, 8)
    wk2 = _combine_bases(comb2, bases2, 128)
    xg2p = _rgcn_dense(agg2, cnt, h1, wk2, root2, bias2, False, 8, 208)

    sflat = sample_index.reshape(B * S)
    en, sam = _gather_sc(xg2p, xclip, index_list, sflat)
    wp = jnp.concatenate([weights, jnp.zeros((8, 1), jnp.float32)]).T
    return _predict(en, sam.reshape(B, S, 208), wp)


# speedup vs baseline: 1.0350x; 1.0350x over previous
import functools

import jax
import jax.numpy as jnp
from jax import lax
from jax.experimental import pallas as pl
from jax.experimental.pallas import tpu as pltpu
from jax.experimental.pallas import tpu_sc as plsc

NC = 2
NS = 16
L = 16

N1 = 12000
N2 = 10000
R = 8
NB = 4
D_IN = 200
D_H = 128
D_F = 200
B = 512
S = 64

_C_EPT = 12288
_C_E1P = _C_EPT * NS
_C_NSB = _C_EPT // 256
_C_ROWS = 12160
_C_RPT = _C_ROWS // NS
_C_W = 64
_C_NP = 4

_G_EPT = 20480
_G_E2P = _G_EPT * NS
_G_NSB = _G_EPT // 256
_G_EPW = _G_E2P // (NC * NS)
_G_NSBW = _G_EPW // 256
_G_ROWS = 80128
_G_RPT = _G_ROWS // NS
_G_W = 16


def _mesh():
    return plsc.VectorSubcoreMesh(core_axis_name="c", subcore_axis_name="s",
                                  num_cores=NC, num_subcores=NS)


def _concept_sc(xt, srcp, dstp, z2d, z16, ones16):
    ppc = _C_NP // NC
    nst = _C_NSB
    half = nst // 2

    @functools.partial(
        pl.kernel,
        out_type=(jax.ShapeDtypeStruct((_C_NP, _C_ROWS, _C_W), jnp.float32),
                  jax.ShapeDtypeStruct((_C_ROWS, 16), jnp.float32)),
        mesh=_mesh(),
        compiler_params=pltpu.CompilerParams(use_tc_tiling_on_sc=False),
        scratch_types=[
            pltpu.VMEM((_C_NSB, 256), jnp.int32),
            pltpu.VMEM((_C_NSB, 256), jnp.int32),
            pltpu.VMEM((256, _C_W), jnp.float32),
            pltpu.VMEM((256, _C_W), jnp.float32),
            pltpu.VMEM((256, 16), jnp.float32),
            pltpu.VMEM_SHARED((_C_ROWS, _C_W), jnp.float32),
            pltpu.VMEM_SHARED((_C_ROWS, 16), jnp.float32),
            pltpu.SemaphoreType.DMA,
            pltpu.SemaphoreType.DMA,
        ],
    )
    def k(xt_h, srcp_h, dstp_h, z2d_h, z16_h, ones_h, out_agg, out_deg,
          segb, gidx, rowsa, rowsb, ones_v, acc, deg, sema, semb):
        c = lax.axis_index("c")
        s = lax.axis_index("s")
        pltpu.sync_copy(ones_h, ones_v)
        r0 = s * _C_RPT
        pltpu.sync_copy(dstp_h.at[pl.ds(s * nst, nst)], segb)

        @pl.when(c == 0)
        def _():
            pltpu.sync_copy(z16_h.at[pl.ds(r0, _C_RPT)], deg.at[pl.ds(r0, _C_RPT)])

        for j in range(ppc):
            p = c * ppc + j
            pltpu.sync_copy(z2d_h.at[pl.ds(r0, _C_RPT)], acc.at[pl.ds(r0, _C_RPT)])
            pltpu.sync_copy(srcp_h.at[pl.ds(s * nst, nst)], gidx)

            def gstep(st, carry):
                for i in range(256 // L):
                    sl = pl.ds(i * L, L)
                    gidx[st, sl] = gidx[st, sl] * _C_NP + p
                return carry

            lax.fori_loop(0, nst, gstep, 0)
            plsc.subcore_barrier()
            pltpu.async_copy(xt_h.at[gidx.at[0]], rowsa, sema)

            def pair(t, carry):
                i0 = 2 * t
                pltpu.async_copy(xt_h.at[gidx.at[i0 + 1]], rowsb, semb)
                pltpu.make_async_copy(xt_h.at[gidx.at[i0]], rowsa, sema).wait()
                pltpu.sync_copy(rowsa, acc.at[segb.at[i0]], add=True)

                @pl.when(t < half - 1)
                def _():
                    pltpu.async_copy(xt_h.at[gidx.at[i0 + 2]], rowsa, sema)

                pltpu.make_async_copy(xt_h.at[gidx.at[i0 + 1]], rowsb, semb).wait()
                pltpu.sync_copy(rowsb, acc.at[segb.at[i0 + 1]], add=True)
                if j == 0:
                    @pl.when(c == 0)
                    def _():
                        pltpu.sync_copy(ones_v, deg.at[segb.at[i0]], add=True)
                        pltpu.sync_copy(ones_v, deg.at[segb.at[i0 + 1]], add=True)
                return carry

            lax.fori_loop(0, half, pair, 0)
            plsc.subcore_barrier()
            pltpu.sync_copy(acc.at[pl.ds(r0, _C_RPT)],
                            out_agg.at[p, pl.ds(r0, _C_RPT)])

        @pl.when(c == 0)
        def _():
            pltpu.sync_copy(deg.at[pl.ds(r0, _C_RPT)], out_deg.at[pl.ds(r0, _C_RPT)])

    return k(xt, srcp, dstp, z2d, z16, ones16)


def _cnt_sc(dstp, typep, z16, ones16):
    @functools.partial(
        pl.kernel,
        out_type=jax.ShapeDtypeStruct((NC, _G_ROWS, 16), jnp.float32),
        mesh=_mesh(),
        compiler_params=pltpu.CompilerParams(use_tc_tiling_on_sc=False),
        scratch_types=[
            pltpu.VMEM((_G_NSBW, 256), jnp.int32),
            pltpu.VMEM((_G_NSBW, 256), jnp.int32),
            pltpu.VMEM((256, 16), jnp.float32),
            pltpu.VMEM_SHARED((_G_ROWS, 16), jnp.float32),
        ],
    )
    def k(dstp_h, typep_h, z16_h, ones_h, out_cnt, segb, tyb, ones_v, cntb):
        c = lax.axis_index("c")
        s = lax.axis_index("s")
        pltpu.sync_copy(ones_h, ones_v)
        g0 = s * _G_RPT
        pltpu.sync_copy(z16_h.at[pl.ds(g0, _G_RPT)], cntb.at[pl.ds(g0, _G_RPT)])
        wrow = (s * NC + c) * _G_NSBW
        pltpu.sync_copy(dstp_h.at[pl.ds(wrow, _G_NSBW)], segb)
        pltpu.sync_copy(typep_h.at[pl.ds(wrow, _G_NSBW)], tyb)

        def prep(st, carry):
            for i in range(256 // L):
                sl = pl.ds(i * L, L)
                segb[st, sl] = tyb[st, sl] * N2 + segb[st, sl]
            return carry

        lax.fori_loop(0, _G_NSBW, prep, 0)
        plsc.subcore_barrier()

        def cstep(st, carry):
            pltpu.sync_copy(ones_v, cntb.at[segb.at[st]], add=True)
            return carry

        lax.fori_loop(0, _G_NSBW, cstep, 0)
        plsc.subcore_barrier()
        pltpu.sync_copy(cntb.at[pl.ds(g0, _G_RPT)], out_cnt.at[c, pl.ds(g0, _G_RPT)])

    return k(dstp, typep, z16, ones16)


def _rgcn_sc(xt, srcp, dstp, typep, z16, np16):
    ppc = np16 // NC
    nst = _G_NSB
    half = nst // 2

    @functools.partial(
        pl.kernel,
        out_type=jax.ShapeDtypeStruct((np16, _G_ROWS, _G_W), jnp.float32),
        mesh=_mesh(),
        compiler_params=pltpu.CompilerParams(use_tc_tiling_on_sc=False),
        scratch_types=[
            pltpu.VMEM((nst, 256), jnp.int32),
            pltpu.VMEM((nst, 256), jnp.int32),
            pltpu.VMEM((256, _G_W), jnp.float32),
            pltpu.VMEM((256, _G_W), jnp.float32),
            pltpu.VMEM_SHARED((_G_ROWS, _G_W), jnp.float32),
            pltpu.SemaphoreType.DMA,
            pltpu.SemaphoreType.DMA,
        ],
    )
    def k(xt_h, srcp_h, dstp_h, typep_h, z16_h,
          out_agg, segb, gidx, rowsa, rowsb, acc, sema, semb):
        c = lax.axis_index("c")
        s = lax.axis_index("s")
        r0 = s * _G_RPT
        pltpu.sync_copy(dstp_h.at[pl.ds(s * nst, nst)], segb)
        pltpu.sync_copy(typep_h.at[pl.ds(s * nst, nst)], gidx)

        def segstep(st, carry):
            for i in range(256 // L):
                sl = pl.ds(i * L, L)
                segb[st, sl] = gidx[st, sl] * N2 + segb[st, sl]
            return carry

        lax.fori_loop(0, nst, segstep, 0)

        for j in range(ppc):
            p = c * ppc + j
            pltpu.sync_copy(z16_h.at[pl.ds(r0, _G_RPT)], acc.at[pl.ds(r0, _G_RPT)])
            pltpu.sync_copy(srcp_h.at[pl.ds(s * nst, nst)], gidx)

            def gstep(st, carry):
                for i in range(256 // L):
                    sl = pl.ds(i * L, L)
                    gidx[st, sl] = gidx[st, sl] * np16 + p
                return carry

            lax.fori_loop(0, nst, gstep, 0)
            plsc.subcore_barrier()

            pltpu.async_copy(xt_h.at[gidx.at[0]], rowsa, sema)

            def pair(t, carry):
                i0 = 2 * t
                pltpu.async_copy(xt_h.at[gidx.at[i0 + 1]], rowsb, semb)
                pltpu.make_async_copy(xt_h.at[gidx.at[i0]], rowsa, sema).wait()
                pltpu.sync_copy(rowsa, acc.at[segb.at[i0]], add=True)

                @pl.when(t < half - 1)
                def _():
                    pltpu.async_copy(xt_h.at[gidx.at[i0 + 2]], rowsa, sema)

                pltpu.make_async_copy(xt_h.at[gidx.at[i0 + 1]], rowsb, semb).wait()
                pltpu.sync_copy(rowsb, acc.at[segb.at[i0 + 1]], add=True)
                return carry

            lax.fori_loop(0, half, pair, 0)
            plsc.subcore_barrier()
            pltpu.sync_copy(acc.at[pl.ds(r0, _G_RPT)],
                            out_agg.at[p, pl.ds(r0, _G_RPT)])

    return k(xt, srcp, dstp, typep, z16)


def _pad_edges_g2(edge_index, edge_type):
    pad = _G_E2P - edge_index.shape[1]
    ar = jnp.arange(pad, dtype=jnp.int32)
    srcp = jnp.concatenate([edge_index[0], jnp.zeros((pad,), jnp.int32)])
    dstp = jnp.concatenate([edge_index[1], ar % 128])
    typep = jnp.concatenate([edge_type, jnp.full((pad,), R, jnp.int32)])
    return (srcp.reshape(-1, 256), dstp.reshape(-1, 256),
            typep.reshape(-1, 256))


def _combine_bases(comb, bases, kp):
    kk, mm = bases.shape[1], bases.shape[2]

    def body(comb_ref, bases_ref, out_ref):
        cb = comb_ref[...]
        bs = bases_ref[...].reshape(NB, kk * mm)
        wf = jnp.dot(cb, bs, preferred_element_type=jnp.float32).reshape(R, kk, mm)
        if kp > kk:
            wf = jnp.concatenate(
                [wf, jnp.zeros((R, kp - kk, mm), jnp.float32)], axis=1)
        out_ref[...] = wf

    return pl.pallas_call(
        body, out_shape=jax.ShapeDtypeStruct((R, kp, mm), jnp.float32),
    )(comb, bases)


def _rgcn_dense(agg, cnt, x, wk, root, bias, relu, np16, mp):
    kk = np16 * _G_W
    mm = wk.shape[2]
    bn = 400
    nblk = N2 // bn

    def body(agg_ref, cnt_ref, x_ref, wk_ref, root_ref, bias_ref, out_ref):
        rr = pl.program_id(1)

        @pl.when(rr < R)
        def _():
            ab = agg_ref[...]
            a = jnp.concatenate([ab[i] for i in range(np16)], axis=1)
            cv = cnt_ref[...]
            a = a / jnp.maximum(cv[0, :, 0:1] + cv[1, :, 0:1], 1.0)
            t = jnp.dot(a, wk_ref[...][0], preferred_element_type=jnp.float32)

            @pl.when(rr == 0)
            def _():
                if mp > mm:
                    out_ref[...] = jnp.concatenate(
                        [t, jnp.zeros((bn, mp - mm), jnp.float32)], axis=1)
                else:
                    out_ref[...] = t

            @pl.when(rr > 0)
            def _():
                out_ref[:, :mm] += t

        @pl.when(rr == R)
        def _():
            t = jnp.dot(x_ref[...], root_ref[...],
                        preferred_element_type=jnp.float32) + bias_ref[...]
            acc2 = out_ref[:, :mm] + t
            if relu:
                acc2 = jax.nn.relu(acc2)
            out_ref[:, :mm] = acc2

    return pl.pallas_call(
        body,
        grid=(nblk, R + 1),
        in_specs=[
            pl.BlockSpec((np16, bn, _G_W),
                         lambda nb, rr: (0, jnp.minimum(rr, R - 1) * (N2 // 400) + nb, 0)),
            pl.BlockSpec((NC, bn, 16),
                         lambda nb, rr: (0, jnp.minimum(rr, R - 1) * (N2 // 400) + nb, 0)),
            pl.BlockSpec((bn, x.shape[1]), lambda nb, rr: (nb, 0)),
            pl.BlockSpec((1, kk, mm), lambda nb, rr: (jnp.minimum(rr, R - 1), 0, 0)),
            pl.BlockSpec((x.shape[1], mm), lambda nb, rr: (0, 0)),
            pl.BlockSpec((1, mm), lambda nb, rr: (0, 0)),
        ],
        out_specs=pl.BlockSpec((bn, mp), lambda nb, rr: (nb, 0)),
        out_shape=jax.ShapeDtypeStruct((N2, mp), jnp.float32),
    )(agg, cnt, x, wk, root, bias.reshape(1, mm))


def _norm_concept(agg, deg):
    bn = 480
    nblk = N1 // bn

    def body(agg_ref, deg_ref, out1_ref, out2_ref):
        ab = agg_ref[...]
        d = jnp.maximum(deg_ref[...][:, 0:1], 1.0)
        xv = jnp.concatenate([ab[i] for i in range(_C_NP)], axis=1)[:, :D_IN]
        xv = jax.nn.relu(xv / d)
        out1_ref[...] = xv
        out2_ref[...] = jnp.concatenate(
            [jnp.clip(xv, 0.0, 1.0), jnp.zeros((bn, 8), jnp.float32)], axis=1)

    return pl.pallas_call(
        body,
        grid=(nblk,),
        in_specs=[
            pl.BlockSpec((_C_NP, bn, _C_W), lambda nb: (0, nb, 0)),
            pl.BlockSpec((bn, 16), lambda nb: (nb, 0)),
        ],
        out_specs=[
            pl.BlockSpec((bn, D_IN), lambda nb: (nb, 0)),
            pl.BlockSpec((bn, 208), lambda nb: (nb, 0)),
        ],
        out_shape=[
            jax.ShapeDtypeStruct((N1, D_IN), jnp.float32),
            jax.ShapeDtypeStruct((N1, 208), jnp.float32),
        ],
    )(agg, deg)


def _concept_layer(x, edge_index):
    src = edge_index[0]
    dst = edge_index[1]
    pad = _C_E1P - src.shape[0]
    srcp = jnp.concatenate([src, jnp.zeros((pad,), jnp.int32)]).reshape(-1, 256)
    dstp = jnp.concatenate(
        [dst, N1 + (jnp.arange(pad, dtype=jnp.int32) % 128)]).reshape(-1, 256)
    xt = jnp.pad(x, ((0, 0), (0, _C_NP * _C_W - D_IN))).reshape(N1 * _C_NP, _C_W)
    z2d = jnp.zeros((_C_ROWS, _C_W), jnp.float32)
    z16c = jnp.zeros((_C_ROWS, 16), jnp.float32)
    ones16 = jnp.ones((256, 16), jnp.float32)
    agg, deg = _concept_sc(xt, srcp, dstp, z2d, z16c, ones16)
    return _norm_concept(agg[:, :N1], deg[:N1])


def _gather_sc(xg2p, xclip, ilist, sflat):
    spt = (B * S) // (NC * NS)
    nstr = spt // 128
    ept = B // (NC * NS)

    @functools.partial(
        pl.kernel,
        out_type=(jax.ShapeDtypeStruct((B, 208), jnp.float32),
                  jax.ShapeDtypeStruct((B * S, 208), jnp.float32)),
        mesh=_mesh(),
        compiler_params=pltpu.CompilerParams(use_tc_tiling_on_sc=False),
        scratch_types=[
            pltpu.VMEM((1, 128), jnp.int32),
            pltpu.VMEM((1, ept), jnp.int32),
            pltpu.VMEM((128, 208), jnp.float32),
            pltpu.VMEM((ept, 208), jnp.float32),
            pltpu.SemaphoreType.DMA,
        ],
    )
    def k(xg2_h, xclip_h, ilist_h, sflat_h, out_en, out_sam,
          sidx, eidx, rows, rows_e, sem):
        c = lax.axis_index("c")
        s = lax.axis_index("s")
        wid = s * NC + c
        pltpu.sync_copy(ilist_h.at[pl.ds(wid * ept, ept)], eidx.at[0])
        pltpu.async_copy(xg2_h.at[eidx.at[0]], rows_e, sem).wait()
        pltpu.sync_copy(rows_e, out_en.at[pl.ds(wid * ept, ept)])

        def step(st, carry):
            off = wid * spt + st * 128
            pltpu.sync_copy(sflat_h.at[pl.ds(off, 128)], sidx.at[0])
            pltpu.async_copy(xclip_h.at[sidx.at[0]], rows, sem).wait()
            pltpu.sync_copy(rows, out_sam.at[pl.ds(off, 128)])
            return carry

        lax.fori_loop(0, nstr, step, 0)

    return k(xg2p, xclip, ilist, sflat)


def _predict_body(en_ref, sam_ref, w_ref, out_ref):
    en = en_ref[...]
    sam = sam_ref[...]
    w = jnp.clip(w_ref[...], 0.0, 1.0)
    v = en * en * w
    tmp = lax.dot_general(sam, v, (((2,), (1,)), ((0,), (0,))),
                          preferred_element_type=jnp.float32)
    m = jnp.max(tmp, axis=1, keepdims=True)
    e = jnp.exp(tmp - m)
    out_ref[...] = e / jnp.sum(e, axis=1, keepdims=True)


def _predict(en, sam, wp):
    bb = 64
    return pl.pallas_call(
        _predict_body,
        grid=(B // bb,),
        in_specs=[
            pl.BlockSpec((bb, 208), lambda i: (i, 0)),
            pl.BlockSpec((bb, S, 208), lambda i: (i, 0, 0)),
            pl.BlockSpec((1, 208), lambda i: (0, 0)),
        ],
        out_specs=pl.BlockSpec((bb, S), lambda i: (i, 0)),
        out_shape=jax.ShapeDtypeStruct((B, S), jnp.float32),
    )(en, sam, wp)


def kernel(all_node_embedding, edge_index_g2, edge_type_g2, edge_index_g1, index_list, sample_index,
           bases1, comb1, root1, bias1, bases2, comb2, root2, bias2, weights):
    x_g1, xclip = _concept_layer(all_node_embedding, edge_index_g1)

    srcp2, dstp2, typep2 = _pad_edges_g2(edge_index_g2, edge_type_g2)
    z16 = jnp.zeros((_G_ROWS, 16), jnp.float32)
    ones16 = jnp.ones((256, 16), jnp.float32)
    cnt = _cnt_sc(dstp2, typep2, z16, ones16)

    x2 = x_g1[:N2]
    xt1 = jnp.pad(x2, ((0, 0), (0, 24))).reshape(N2 * 14, _G_W)
    agg1 = _rgcn_sc(xt1, srcp2, dstp2, typep2, z16, 14)
    wk1 = _combine_bases(comb1, bases1, 224)
    h1 = _rgcn_dense(agg1, cnt, x2, wk1, root1, bias1, True, 14, D_H)

    xt2 = h1.reshape(N2 * 8, _G_W)
    agg2 = _rgcn_sc(xt2, srcp2, dstp2, typep2, z16, 8)
    wk2 = _combine_bases(comb2, bases2, 128)
    xg2p = _rgcn_dense(agg2, cnt, h1, wk2, root2, bias2, False, 8, 208)

    sflat = sample_index.reshape(B * S)
    en, sam = _gather_sc(xg2p, xclip, index_list, sflat)
    wp = jnp.concatenate([weights, jnp.zeros((8, 1), jnp.float32)]).T
    return _predict(en, sam.reshape(B, S, 208), wp)


# speedup vs baseline: 1.0379x; 1.0029x over previous
import functools

import jax
import jax.numpy as jnp
from jax import lax
from jax.experimental import pallas as pl
from jax.experimental.pallas import tpu as pltpu
from jax.experimental.pallas import tpu_sc as plsc

NC = 2
NS = 16
L = 16

N1 = 12000
N2 = 10000
R = 8
NB = 4
D_IN = 200
D_H = 128
D_F = 200
B = 512
S = 64

_C_EPT = 12288
_C_E1P = _C_EPT * NS
_C_NSB = _C_EPT // 256
_C_ROWS = 12160
_C_RPT = _C_ROWS // NS
_C_W = 64
_C_NP = 4

_G_EPT = 20480
_G_E2P = _G_EPT * NS
_G_NSB = _G_EPT // 256
_G_EPW = _G_E2P // (NC * NS)
_G_NSBW = _G_EPW // 256
_G_ROWS = 80128
_G_RPT = _G_ROWS // NS
_G_W = 16


def _mesh():
    return plsc.VectorSubcoreMesh(core_axis_name="c", subcore_axis_name="s",
                                  num_cores=NC, num_subcores=NS)


def _concept_sc(xt, srcp, dstp, z2d, z16, ones16):
    ppc = _C_NP // NC
    nst = _C_NSB
    half = nst // 2

    @functools.partial(
        pl.kernel,
        out_type=(jax.ShapeDtypeStruct((_C_NP, _C_ROWS, _C_W), jnp.float32),
                  jax.ShapeDtypeStruct((_C_ROWS, 16), jnp.float32)),
        mesh=_mesh(),
        compiler_params=pltpu.CompilerParams(use_tc_tiling_on_sc=False),
        scratch_types=[
            pltpu.VMEM((_C_NSB, 256), jnp.int32),
            pltpu.VMEM((_C_NSB, 256), jnp.int32),
            pltpu.VMEM((256, _C_W), jnp.float32),
            pltpu.VMEM((256, _C_W), jnp.float32),
            pltpu.VMEM((256, 16), jnp.float32),
            pltpu.VMEM_SHARED((_C_ROWS, _C_W), jnp.float32),
            pltpu.VMEM_SHARED((_C_ROWS, 16), jnp.float32),
            pltpu.SemaphoreType.DMA,
            pltpu.SemaphoreType.DMA,
        ],
    )
    def k(xt_h, srcp_h, dstp_h, z2d_h, z16_h, ones_h, out_agg, out_deg,
          segb, gidx, rowsa, rowsb, ones_v, acc, deg, sema, semb):
        c = lax.axis_index("c")
        s = lax.axis_index("s")
        pltpu.sync_copy(ones_h, ones_v)
        r0 = s * _C_RPT
        pltpu.sync_copy(dstp_h.at[pl.ds(s * nst, nst)], segb)

        @pl.when(c == 0)
        def _():
            pltpu.sync_copy(z16_h.at[pl.ds(r0, _C_RPT)], deg.at[pl.ds(r0, _C_RPT)])

        for j in range(ppc):
            p = c * ppc + j
            pltpu.sync_copy(z2d_h.at[pl.ds(r0, _C_RPT)], acc.at[pl.ds(r0, _C_RPT)])
            pltpu.sync_copy(srcp_h.at[pl.ds(s * nst, nst)], gidx)

            def gstep(st, carry):
                for i in range(256 // L):
                    sl = pl.ds(i * L, L)
                    gidx[st, sl] = gidx[st, sl] * _C_NP + p
                return carry

            lax.fori_loop(0, nst, gstep, 0)
            plsc.subcore_barrier()
            pltpu.async_copy(xt_h.at[gidx.at[0]], rowsa, sema)

            def pair(t, carry):
                i0 = 2 * t
                pltpu.async_copy(xt_h.at[gidx.at[i0 + 1]], rowsb, semb)
                pltpu.make_async_copy(xt_h.at[gidx.at[i0]], rowsa, sema).wait()
                pltpu.sync_copy(rowsa, acc.at[segb.at[i0]], add=True)

                @pl.when(t < half - 1)
                def _():
                    pltpu.async_copy(xt_h.at[gidx.at[i0 + 2]], rowsa, sema)

                pltpu.make_async_copy(xt_h.at[gidx.at[i0 + 1]], rowsb, semb).wait()
                pltpu.sync_copy(rowsb, acc.at[segb.at[i0 + 1]], add=True)
                if j == 0:
                    @pl.when(c == 0)
                    def _():
                        pltpu.sync_copy(ones_v, deg.at[segb.at[i0]], add=True)
                        pltpu.sync_copy(ones_v, deg.at[segb.at[i0 + 1]], add=True)
                return carry

            lax.fori_loop(0, half, pair, 0)
            plsc.subcore_barrier()
            pltpu.sync_copy(acc.at[pl.ds(r0, _C_RPT)],
                            out_agg.at[p, pl.ds(r0, _C_RPT)])

        @pl.when(c == 0)
        def _():
            pltpu.sync_copy(deg.at[pl.ds(r0, _C_RPT)], out_deg.at[pl.ds(r0, _C_RPT)])

    return k(xt, srcp, dstp, z2d, z16, ones16)


def _cnt_sc(dstp, typep, z16, ones16):
    @functools.partial(
        pl.kernel,
        out_type=jax.ShapeDtypeStruct((NC, _G_ROWS, 16), jnp.float32),
        mesh=_mesh(),
        compiler_params=pltpu.CompilerParams(use_tc_tiling_on_sc=False),
        scratch_types=[
            pltpu.VMEM((_G_NSBW, 256), jnp.int32),
            pltpu.VMEM((_G_NSBW, 256), jnp.int32),
            pltpu.VMEM((256, 16), jnp.float32),
            pltpu.VMEM_SHARED((_G_ROWS, 16), jnp.float32),
        ],
    )
    def k(dstp_h, typep_h, z16_h, ones_h, out_cnt, segb, tyb, ones_v, cntb):
        c = lax.axis_index("c")
        s = lax.axis_index("s")
        pltpu.sync_copy(ones_h, ones_v)
        g0 = s * _G_RPT
        pltpu.sync_copy(z16_h.at[pl.ds(g0, _G_RPT)], cntb.at[pl.ds(g0, _G_RPT)])
        wrow = (s * NC + c) * _G_NSBW
        pltpu.sync_copy(dstp_h.at[pl.ds(wrow, _G_NSBW)], segb)
        pltpu.sync_copy(typep_h.at[pl.ds(wrow, _G_NSBW)], tyb)

        def prep(st, carry):
            for i in range(256 // L):
                sl = pl.ds(i * L, L)
                segb[st, sl] = tyb[st, sl] * N2 + segb[st, sl]
            return carry

        lax.fori_loop(0, _G_NSBW, prep, 0)
        plsc.subcore_barrier()

        def cstep(st, carry):
            pltpu.sync_copy(ones_v, cntb.at[segb.at[st]], add=True)
            return carry

        lax.fori_loop(0, _G_NSBW, cstep, 0)
        plsc.subcore_barrier()
        pltpu.sync_copy(cntb.at[pl.ds(g0, _G_RPT)], out_cnt.at[c, pl.ds(g0, _G_RPT)])

    return k(dstp, typep, z16, ones16)


def _rgcn_sc(xt, srcp, dstp, typep, z16, np16):
    ppc = np16 // NC
    nst = _G_NSB
    half = nst // 2

    @functools.partial(
        pl.kernel,
        out_type=jax.ShapeDtypeStruct((np16, _G_ROWS, _G_W), jnp.float32),
        mesh=_mesh(),
        compiler_params=pltpu.CompilerParams(use_tc_tiling_on_sc=False),
        scratch_types=[
            pltpu.VMEM((nst, 256), jnp.int32),
            pltpu.VMEM((nst, 256), jnp.int32),
            pltpu.VMEM((256, _G_W), jnp.float32),
            pltpu.VMEM((256, _G_W), jnp.float32),
            pltpu.VMEM_SHARED((_G_ROWS, _G_W), jnp.float32),
            pltpu.SemaphoreType.DMA,
            pltpu.SemaphoreType.DMA,
        ],
    )
    def k(xt_h, srcp_h, dstp_h, typep_h, z16_h,
          out_agg, segb, gidx, rowsa, rowsb, acc, sema, semb):
        c = lax.axis_index("c")
        s = lax.axis_index("s")
        r0 = s * _G_RPT
        pltpu.sync_copy(dstp_h.at[pl.ds(s * nst, nst)], segb)
        pltpu.sync_copy(typep_h.at[pl.ds(s * nst, nst)], gidx)

        def segstep(st, carry):
            for i in range(256 // L):
                sl = pl.ds(i * L, L)
                segb[st, sl] = gidx[st, sl] * N2 + segb[st, sl]
            return carry

        lax.fori_loop(0, nst, segstep, 0)

        for j in range(ppc):
            p = c * ppc + j
            pltpu.sync_copy(z16_h.at[pl.ds(r0, _G_RPT)], acc.at[pl.ds(r0, _G_RPT)])
            pltpu.sync_copy(srcp_h.at[pl.ds(s * nst, nst)], gidx)

            def gstep(st, carry):
                for i in range(256 // L):
                    sl = pl.ds(i * L, L)
                    gidx[st, sl] = gidx[st, sl] * np16 + p
                return carry

            lax.fori_loop(0, nst, gstep, 0)
            plsc.subcore_barrier()

            pltpu.async_copy(xt_h.at[gidx.at[0]], rowsa, sema)

            def pair(t, carry):
                i0 = 2 * t
                pltpu.async_copy(xt_h.at[gidx.at[i0 + 1]], rowsb, semb)
                pltpu.make_async_copy(xt_h.at[gidx.at[i0]], rowsa, sema).wait()
                pltpu.sync_copy(rowsa, acc.at[segb.at[i0]], add=True)

                @pl.when(t < half - 1)
                def _():
                    pltpu.async_copy(xt_h.at[gidx.at[i0 + 2]], rowsa, sema)

                pltpu.make_async_copy(xt_h.at[gidx.at[i0 + 1]], rowsb, semb).wait()
                pltpu.sync_copy(rowsb, acc.at[segb.at[i0 + 1]], add=True)
                return carry

            lax.fori_loop(0, half, pair, 0)
            plsc.subcore_barrier()
            pltpu.sync_copy(acc.at[pl.ds(r0, _G_RPT)],
                            out_agg.at[p, pl.ds(r0, _G_RPT)])

    return k(xt, srcp, dstp, typep, z16)


def _pad_edges_g2(edge_index, edge_type):
    pad = _G_E2P - edge_index.shape[1]
    ar = jnp.arange(pad, dtype=jnp.int32)
    srcp = jnp.concatenate([edge_index[0], jnp.zeros((pad,), jnp.int32)])
    dstp = jnp.concatenate([edge_index[1], ar % 128])
    typep = jnp.concatenate([edge_type, jnp.full((pad,), R, jnp.int32)])
    return (srcp.reshape(-1, 256), dstp.reshape(-1, 256),
            typep.reshape(-1, 256))


def _combine_bases(comb, bases, kp):
    kk, mm = bases.shape[1], bases.shape[2]

    def body(comb_ref, bases_ref, out_ref):
        cb = comb_ref[...]
        bs = bases_ref[...].reshape(NB, kk * mm)
        wf = jnp.dot(cb, bs, preferred_element_type=jnp.float32).reshape(R, kk, mm)
        if kp > kk:
            wf = jnp.concatenate(
                [wf, jnp.zeros((R, kp - kk, mm), jnp.float32)], axis=1)
        out_ref[...] = wf

    return pl.pallas_call(
        body, out_shape=jax.ShapeDtypeStruct((R, kp, mm), jnp.float32),
    )(comb, bases)


def _rgcn_dense(agg, cnt, x, comb, bases, root, bias, relu, np16, mp):
    kk = np16 * _G_W
    kt = bases.shape[1]
    mm = bases.shape[2]
    bn = 400
    nblk = N2 // bn

    def body(agg_ref, cnt_ref, x_ref, comb_ref, bases_ref, root_ref, bias_ref, out_ref):
        rr = pl.program_id(1)

        @pl.when(rr < R)
        def _():
            ab = agg_ref[...]
            a = jnp.concatenate([ab[i] for i in range(np16)], axis=1)
            cv = cnt_ref[...]
            a = a / jnp.maximum(cv[0, :, 0:1] + cv[1, :, 0:1], 1.0)
            onehot = (lax.broadcasted_iota(jnp.int32, (R, NB), 0) == rr
                      ).astype(jnp.float32)
            coef = jnp.sum(comb_ref[...] * onehot, axis=0)
            bs = bases_ref[...]
            wr = (coef[0] * bs[0] + coef[1] * bs[1]
                  + coef[2] * bs[2] + coef[3] * bs[3])
            t = jnp.dot(a[:, :kt], wr, preferred_element_type=jnp.float32)

            @pl.when(rr == 0)
            def _():
                if mp > mm:
                    out_ref[...] = jnp.concatenate(
                        [t, jnp.zeros((bn, mp - mm), jnp.float32)], axis=1)
                else:
                    out_ref[...] = t

            @pl.when(rr > 0)
            def _():
                out_ref[:, :mm] += t

        @pl.when(rr == R)
        def _():
            t = jnp.dot(x_ref[...], root_ref[...],
                        preferred_element_type=jnp.float32) + bias_ref[...]
            acc2 = out_ref[:, :mm] + t
            if relu:
                acc2 = jax.nn.relu(acc2)
            out_ref[:, :mm] = acc2

    return pl.pallas_call(
        body,
        grid=(nblk, R + 1),
        in_specs=[
            pl.BlockSpec((np16, bn, _G_W),
                         lambda nb, rr: (0, jnp.minimum(rr, R - 1) * (N2 // 400) + nb, 0)),
            pl.BlockSpec((NC, bn, 16),
                         lambda nb, rr: (0, jnp.minimum(rr, R - 1) * (N2 // 400) + nb, 0)),
            pl.BlockSpec((bn, x.shape[1]), lambda nb, rr: (nb, 0)),
            pl.BlockSpec((R, NB), lambda nb, rr: (0, 0)),
            pl.BlockSpec((NB, kt, mm), lambda nb, rr: (0, 0, 0)),
            pl.BlockSpec((x.shape[1], mm), lambda nb, rr: (0, 0)),
            pl.BlockSpec((1, mm), lambda nb, rr: (0, 0)),
        ],
        out_specs=pl.BlockSpec((bn, mp), lambda nb, rr: (nb, 0)),
        out_shape=jax.ShapeDtypeStruct((N2, mp), jnp.float32),
    )(agg, cnt, x, comb, bases, root, bias.reshape(1, mm))


def _norm_concept(agg, deg):
    bn = 480
    nblk = N1 // bn

    def body(agg_ref, deg_ref, out1_ref, out2_ref):
        ab = agg_ref[...]
        d = jnp.maximum(deg_ref[...][:, 0:1], 1.0)
        xv = jnp.concatenate([ab[i] for i in range(_C_NP)], axis=1)[:, :D_IN]
        xv = jax.nn.relu(xv / d)
        out1_ref[...] = xv
        out2_ref[...] = jnp.concatenate(
            [jnp.clip(xv, 0.0, 1.0), jnp.zeros((bn, 8), jnp.float32)], axis=1)

    return pl.pallas_call(
        body,
        grid=(nblk,),
        in_specs=[
            pl.BlockSpec((_C_NP, bn, _C_W), lambda nb: (0, nb, 0)),
            pl.BlockSpec((bn, 16), lambda nb: (nb, 0)),
        ],
        out_specs=[
            pl.BlockSpec((bn, D_IN), lambda nb: (nb, 0)),
            pl.BlockSpec((bn, 208), lambda nb: (nb, 0)),
        ],
        out_shape=[
            jax.ShapeDtypeStruct((N1, D_IN), jnp.float32),
            jax.ShapeDtypeStruct((N1, 208), jnp.float32),
        ],
    )(agg, deg)


def _concept_layer(x, edge_index):
    src = edge_index[0]
    dst = edge_index[1]
    pad = _C_E1P - src.shape[0]
    srcp = jnp.concatenate([src, jnp.zeros((pad,), jnp.int32)]).reshape(-1, 256)
    dstp = jnp.concatenate(
        [dst, N1 + (jnp.arange(pad, dtype=jnp.int32) % 128)]).reshape(-1, 256)
    xt = jnp.pad(x, ((0, 0), (0, _C_NP * _C_W - D_IN))).reshape(N1 * _C_NP, _C_W)
    z2d = jnp.zeros((_C_ROWS, _C_W), jnp.float32)
    z16c = jnp.zeros((_C_ROWS, 16), jnp.float32)
    ones16 = jnp.ones((256, 16), jnp.float32)
    agg, deg = _concept_sc(xt, srcp, dstp, z2d, z16c, ones16)
    return _norm_concept(agg[:, :N1], deg[:N1])


def _gather_sc(xg2p, xclip, ilist, sflat):
    spt = (B * S) // (NC * NS)
    nstr = spt // 128
    ept = B // (NC * NS)

    @functools.partial(
        pl.kernel,
        out_type=(jax.ShapeDtypeStruct((B, 208), jnp.float32),
                  jax.ShapeDtypeStruct((B * S, 208), jnp.float32)),
        mesh=_mesh(),
        compiler_params=pltpu.CompilerParams(use_tc_tiling_on_sc=False),
        scratch_types=[
            pltpu.VMEM((1, 128), jnp.int32),
            pltpu.VMEM((1, ept), jnp.int32),
            pltpu.VMEM((128, 208), jnp.float32),
            pltpu.VMEM((ept, 208), jnp.float32),
            pltpu.SemaphoreType.DMA,
        ],
    )
    def k(xg2_h, xclip_h, ilist_h, sflat_h, out_en, out_sam,
          sidx, eidx, rows, rows_e, sem):
        c = lax.axis_index("c")
        s = lax.axis_index("s")
        wid = s * NC + c
        pltpu.sync_copy(ilist_h.at[pl.ds(wid * ept, ept)], eidx.at[0])
        pltpu.async_copy(xg2_h.at[eidx.at[0]], rows_e, sem).wait()
        pltpu.sync_copy(rows_e, out_en.at[pl.ds(wid * ept, ept)])

        def step(st, carry):
            off = wid * spt + st * 128
            pltpu.sync_copy(sflat_h.at[pl.ds(off, 128)], sidx.at[0])
            pltpu.async_copy(xclip_h.at[sidx.at[0]], rows, sem).wait()
            pltpu.sync_copy(rows, out_sam.at[pl.ds(off, 128)])
            return carry

        lax.fori_loop(0, nstr, step, 0)

    return k(xg2p, xclip, ilist, sflat)


def _predict_body(en_ref, sam_ref, w_ref, out_ref):
    en = en_ref[...]
    sam = sam_ref[...]
    w = jnp.clip(w_ref[...], 0.0, 1.0)
    v = en * en * w
    tmp = lax.dot_general(sam, v, (((2,), (1,)), ((0,), (0,))),
                          preferred_element_type=jnp.float32)
    m = jnp.max(tmp, axis=1, keepdims=True)
    e = jnp.exp(tmp - m)
    out_ref[...] = e / jnp.sum(e, axis=1, keepdims=True)


def _predict(en, sam, wp):
    bb = 64
    return pl.pallas_call(
        _predict_body,
        grid=(B // bb,),
        in_specs=[
            pl.BlockSpec((bb, 208), lambda i: (i, 0)),
            pl.BlockSpec((bb, S, 208), lambda i: (i, 0, 0)),
            pl.BlockSpec((1, 208), lambda i: (0, 0)),
        ],
        out_specs=pl.BlockSpec((bb, S), lambda i: (i, 0)),
        out_shape=jax.ShapeDtypeStruct((B, S), jnp.float32),
    )(en, sam, wp)


def kernel(all_node_embedding, edge_index_g2, edge_type_g2, edge_index_g1, index_list, sample_index,
           bases1, comb1, root1, bias1, bases2, comb2, root2, bias2, weights):
    x_g1, xclip = _concept_layer(all_node_embedding, edge_index_g1)

    srcp2, dstp2, typep2 = _pad_edges_g2(edge_index_g2, edge_type_g2)
    z16 = jnp.zeros((_G_ROWS, 16), jnp.float32)
    ones16 = jnp.ones((256, 16), jnp.float32)
    cnt = _cnt_sc(dstp2, typep2, z16, ones16)

    x2 = x_g1[:N2]
    xt1 = jnp.pad(x2, ((0, 0), (0, 24))).reshape(N2 * 14, _G_W)
    agg1 = _rgcn_sc(xt1, srcp2, dstp2, typep2, z16, 14)
    h1 = _rgcn_dense(agg1, cnt, x2, comb1, bases1, root1, bias1, True, 14, D_H)

    xt2 = h1.reshape(N2 * 8, _G_W)
    agg2 = _rgcn_sc(xt2, srcp2, dstp2, typep2, z16, 8)
    xg2p = _rgcn_dense(agg2, cnt, h1, comb2, bases2, root2, bias2, False, 8, 208)

    sflat = sample_index.reshape(B * S)
    en, sam = _gather_sc(xg2p, xclip, index_list, sflat)
    wp = jnp.concatenate([weights, jnp.zeros((8, 1), jnp.float32)]).T
    return _predict(en, sam.reshape(B, S, 208), wp)
